# baseline (speedup 1.0000x reference)
import jax
import jax.numpy as jnp
from jax import lax
from jax.experimental import pallas as pl
from jax.experimental.pallas import tpu as pltpu
from jax.experimental.pallas import tpu_sc as plsc

VOCAB = 8192
B, T = 16, 512
N_TOK = B * T

_info = plsc.get_sparse_core_info()
NC, NS = _info.num_cores, _info.num_subcores
NW = NC * NS
TOK_PER_W = N_TOK // NW
CH = 8
NCHUNK = TOK_PER_W // CH


def _gather_body(idx_hbm, table_hbm, out_hbm, idx_v, rows_v, sem):
    wid = lax.axis_index("s") * NC + lax.axis_index("c")
    base = wid * TOK_PER_W
    pltpu.sync_copy(idx_hbm.at[pl.ds(base, TOK_PER_W)], idx_v)

    def chunk(u, carry):
        off = u * CH
        pltpu.async_copy(
            table_hbm.at[idx_v.at[pl.ds(off, CH)]], rows_v, sem
        ).wait()
        pltpu.sync_copy(rows_v, out_hbm.at[pl.ds(base + off, CH)])
        return carry

    lax.fori_loop(0, NCHUNK, chunk, 0)


@jax.jit
def _gather(idx_flat, table):
    mesh = plsc.VectorSubcoreMesh(core_axis_name="c", subcore_axis_name="s")
    return pl.kernel(
        _gather_body,
        out_type=jax.ShapeDtypeStruct((N_TOK, VOCAB), jnp.float32),
        mesh=mesh,
        scratch_types=[
            pltpu.VMEM((TOK_PER_W,), jnp.int32),
            pltpu.VMEM((CH, VOCAB), jnp.float32),
            pltpu.SemaphoreType.DMA,
        ],
    )(idx_flat, table)


def kernel(idx, table):
    idx_flat = idx.reshape(N_TOK).astype(jnp.int32)
    out = _gather(idx_flat, table)
    return out.reshape(B, T, VOCAB)
